# baseline (speedup 1.0000x reference)
import functools

import jax
import jax.numpy as jnp
from jax import lax
from jax.experimental import pallas as pl
from jax.experimental.pallas import tpu as pltpu
from jax.experimental.pallas import tpu_sc as plsc

_VOCAB = 1000
_EMB = 128
_B = 16384
_L = 20
_EPS = 1e-5

_NC = 2
_NS = 16
_NW = _NC * _NS
_BPW = _B // _NW
_RB = 4
_NCHUNK = _BPW // _RB
_NBUF = 4
_AHEAD = 2
_ROWS_PER_SUB = 64
_TROWS = 32
_LANES = 16
_NV = _EMB // _LANES


@functools.partial(
    pl.kernel,
    out_type=jax.ShapeDtypeStruct((_B, _L, _EMB), jnp.float32),
    mesh=plsc.VectorSubcoreMesh(core_axis_name="c", subcore_axis_name="s"),
    compiler_params=pltpu.CompilerParams(needs_layout_passes=False),
    scratch_types=[
        pltpu.VMEM((_BPW, _L), jnp.int32),
        pltpu.VMEM((_NBUF, _RB, _L, _EMB), jnp.float32),
        pltpu.VMEM((_TROWS, _EMB), jnp.float32),
        pltpu.VMEM((_EMB,), jnp.float32),
        pltpu.VMEM((_EMB,), jnp.float32),
        pltpu.VMEM_SHARED((_VOCAB, _EMB), jnp.float32),
        pltpu.SemaphoreType.DMA((_NBUF,)),
        pltpu.SemaphoreType.DMA((_NBUF,)),
    ],
)
def _sc_embed(table_hbm, gamma_hbm, beta_hbm, idx_hbm, out_hbm,
              idx_v, bufs, tbl_v, gamma_v, beta_v, ntable_sp, gsems, wsems):
    sid = lax.axis_index("s")
    wid = sid * _NC + lax.axis_index("c")
    base = wid * _BPW

    start = lax.min(sid * _ROWS_PER_SUB, _VOCAB - _ROWS_PER_SUB)
    pltpu.sync_copy(gamma_hbm, gamma_v)
    pltpu.sync_copy(beta_hbm, beta_v)

    inv_emb = 1.0 / _EMB

    def ln_row(i, carry):
        x = [tbl_v[i, pl.ds(j * _LANES, _LANES)] for j in range(_NV)]
        tot = x[0]
        for j in range(1, _NV):
            tot = tot + x[j]
        mean = jnp.full((_LANES,), jnp.sum(tot)) * inv_emb
        c = [xj - mean for xj in x]
        sq = c[0] * c[0]
        for j in range(1, _NV):
            sq = sq + c[j] * c[j]
        var = jnp.full((_LANES,), jnp.sum(sq)) * inv_emb
        v = var + _EPS
        iv = lax.bitcast_convert_type(v, jnp.int32)
        y = lax.bitcast_convert_type(
            jnp.full((_LANES,), jnp.int32(0x5F3759DF)) - (iv >> 1), jnp.float32
        )
        for _ in range(4):
            y = y * (1.5 - 0.5 * v * y * y)
        for j in range(_NV):
            g = gamma_v[pl.ds(j * _LANES, _LANES)]
            b = beta_v[pl.ds(j * _LANES, _LANES)]
            tbl_v[i, pl.ds(j * _LANES, _LANES)] = c[j] * y * g + b
        return carry

    for h in range(_ROWS_PER_SUB // _TROWS):
        hs = start + h * _TROWS
        pltpu.sync_copy(table_hbm.at[pl.ds(hs, _TROWS)], tbl_v)
        lax.fori_loop(0, _TROWS, ln_row, 0)
        pltpu.sync_copy(tbl_v, ntable_sp.at[pl.ds(hs, _TROWS)])
    plsc.subcore_barrier()

    pltpu.sync_copy(idx_hbm.at[pl.ds(base, _BPW)], idx_v)

    def fire_gathers(m, k):
        for j in range(_RB):
            r = m * _RB + j
            pltpu.async_copy(
                ntable_sp.at[idx_v.at[r]], bufs.at[k, j], gsems.at[k]
            )

    def drain_gathers(m, k):
        for j in range(_RB):
            r = m * _RB + j
            pltpu.make_async_copy(
                ntable_sp.at[idx_v.at[r]], bufs.at[k, j], gsems.at[k]
            ).wait()

    def fire_write(m, k):
        pltpu.async_copy(
            bufs.at[k], out_hbm.at[pl.ds(base + m * _RB, _RB)], wsems.at[k]
        )

    def wait_write(m, k):
        pltpu.make_async_copy(
            bufs.at[k], out_hbm.at[pl.ds(base + m * _RB, _RB)], wsems.at[k]
        ).wait()

    for m in range(_AHEAD):
        fire_gathers(m, m % _NBUF)

    def body(m, carry):
        k = m % _NBUF
        drain_gathers(m, k)
        fire_write(m, k)

        nxt = m + _AHEAD
        kn = nxt % _NBUF

        @pl.when(nxt < _NCHUNK)
        def _():
            @pl.when(nxt >= _NBUF)
            def _():
                wait_write(nxt - _NBUF, kn)

            fire_gathers(nxt, kn)

        return carry

    lax.fori_loop(0, _NCHUNK, body, 0)

    for t in range(_NBUF):
        m = _NCHUNK - _NBUF + t
        wait_write(m, m % _NBUF)


def kernel(indices, table, gamma, beta):
    return _sc_embed(table, gamma, beta, indices.astype(jnp.int32))


# speedup vs baseline: 2.2559x; 2.2559x over previous
import functools

import jax
import jax.numpy as jnp
from jax import lax
from jax.experimental import pallas as pl
from jax.experimental.pallas import tpu as pltpu
from jax.experimental.pallas import tpu_sc as plsc

_VOCAB = 1000
_EMB = 128
_B = 16384
_L = 20
_EPS = 1e-5

_NC = 2
_NS = 16
_NW = _NC * _NS
_BPW = _B // _NW
_CW = 128
_CPL = _BPW // _CW
_NCHUNK = _L * _CPL
_NBUF = 4
_AHEAD = 2
_ROWS_PER_SUB = 64
_TROWS = 32
_LANES = 16
_NV = _EMB // _LANES


@functools.partial(
    pl.kernel,
    out_type=jax.ShapeDtypeStruct((_L, _B, _EMB), jnp.float32),
    mesh=plsc.VectorSubcoreMesh(core_axis_name="c", subcore_axis_name="s"),
    compiler_params=pltpu.CompilerParams(needs_layout_passes=False),
    scratch_types=[
        pltpu.VMEM((_L, _BPW), jnp.int32),
        pltpu.VMEM((_NBUF, _CW, _EMB), jnp.float32),
        pltpu.VMEM((_TROWS, _EMB), jnp.float32),
        pltpu.VMEM((_EMB,), jnp.float32),
        pltpu.VMEM((_EMB,), jnp.float32),
        pltpu.VMEM_SHARED((_VOCAB, _EMB), jnp.float32),
        pltpu.SemaphoreType.DMA((_NBUF,)),
        pltpu.SemaphoreType.DMA((_NBUF,)),
    ],
)
def _sc_embed(table_hbm, gamma_hbm, beta_hbm, idx_hbm, out_hbm,
              idx_v, bufs, tbl_v, gamma_v, beta_v, ntable_sp, gsems, wsems):
    sid = lax.axis_index("s")
    wid = sid * _NC + lax.axis_index("c")
    base = wid * _BPW

    start = lax.min(sid * _ROWS_PER_SUB, _VOCAB - _ROWS_PER_SUB)
    pltpu.sync_copy(gamma_hbm, gamma_v)
    pltpu.sync_copy(beta_hbm, beta_v)

    inv_emb = 1.0 / _EMB

    def ln_row(i, carry):
        x = [tbl_v[i, pl.ds(j * _LANES, _LANES)] for j in range(_NV)]
        tot = x[0]
        for j in range(1, _NV):
            tot = tot + x[j]
        mean = jnp.full((_LANES,), jnp.sum(tot)) * inv_emb
        c = [xj - mean for xj in x]
        sq = c[0] * c[0]
        for j in range(1, _NV):
            sq = sq + c[j] * c[j]
        var = jnp.full((_LANES,), jnp.sum(sq)) * inv_emb
        v = var + _EPS
        iv = lax.bitcast_convert_type(v, jnp.int32)
        y = lax.bitcast_convert_type(
            jnp.full((_LANES,), jnp.int32(0x5F3759DF)) - (iv >> 1), jnp.float32
        )
        for _ in range(4):
            y = y * (1.5 - 0.5 * v * y * y)
        for j in range(_NV):
            g = gamma_v[pl.ds(j * _LANES, _LANES)]
            b = beta_v[pl.ds(j * _LANES, _LANES)]
            tbl_v[i, pl.ds(j * _LANES, _LANES)] = c[j] * y * g + b
        return carry

    for h in range(_ROWS_PER_SUB // _TROWS):
        hs = start + h * _TROWS
        pltpu.sync_copy(table_hbm.at[pl.ds(hs, _TROWS)], tbl_v)
        lax.fori_loop(0, _TROWS, ln_row, 0)
        pltpu.sync_copy(tbl_v, ntable_sp.at[pl.ds(hs, _TROWS)])
    plsc.subcore_barrier()

    pltpu.sync_copy(idx_hbm.at[pl.ds(0, _L), pl.ds(base, _BPW)], idx_v)

    def chunk_refs(m, k):
        l = m // _CPL
        col = base + (m % _CPL) * _CW
        idx_sl = idx_v.at[l, pl.ds((m % _CPL) * _CW, _CW)]
        return ntable_sp.at[idx_sl], bufs.at[k], out_hbm.at[l, pl.ds(col, _CW)]

    def fire_gather(m, k):
        src, buf, _ = chunk_refs(m, k)
        pltpu.async_copy(src, buf, gsems.at[k])

    def drain_gather(m, k):
        src, buf, _ = chunk_refs(m, k)
        pltpu.make_async_copy(src, buf, gsems.at[k]).wait()

    def fire_write(m, k):
        _, buf, dst = chunk_refs(m, k)
        pltpu.async_copy(buf, dst, wsems.at[k])

    def wait_write(m, k):
        _, buf, dst = chunk_refs(m, k)
        pltpu.make_async_copy(buf, dst, wsems.at[k]).wait()

    for m in range(_AHEAD):
        fire_gather(m, m % _NBUF)

    def body(m, carry):
        k = m % _NBUF
        drain_gather(m, k)
        fire_write(m, k)

        nxt = m + _AHEAD
        kn = nxt % _NBUF

        @pl.when(nxt < _NCHUNK)
        def _():
            @pl.when(nxt >= _NBUF)
            def _():
                wait_write(nxt - _NBUF, kn)

            fire_gather(nxt, kn)

        return carry

    lax.fori_loop(0, _NCHUNK, body, 0)

    for t in range(_NBUF):
        m = _NCHUNK - _NBUF + t
        wait_write(m, m % _NBUF)


def kernel(indices, table, gamma, beta):
    out_lbe = _sc_embed(table, gamma, beta,
                        jnp.transpose(indices.astype(jnp.int32), (1, 0)))
    return jnp.transpose(out_lbe, (1, 0, 2))


# speedup vs baseline: 2.2898x; 1.0150x over previous
import functools

import jax
import jax.numpy as jnp
from jax import lax
from jax.experimental import pallas as pl
from jax.experimental.pallas import tpu as pltpu
from jax.experimental.pallas import tpu_sc as plsc

_VOCAB = 1000
_EMB = 128
_B = 16384
_L = 20
_EPS = 1e-5

_NC = 2
_NS = 16
_NW = _NC * _NS
_BPW = _B // _NW
_CW = 128
_CPL = _BPW // _CW
_NCHUNK = _L * _CPL
_NBUF = 6
_AHEAD = 3
_ROWS_PER_SUB = 64
_TROWS = 32
_LANES = 16
_NV = _EMB // _LANES


@functools.partial(
    pl.kernel,
    out_type=jax.ShapeDtypeStruct((_L, _B, _EMB), jnp.float32),
    mesh=plsc.VectorSubcoreMesh(core_axis_name="c", subcore_axis_name="s"),
    compiler_params=pltpu.CompilerParams(needs_layout_passes=False),
    scratch_types=[
        pltpu.VMEM((_L, _BPW), jnp.int32),
        pltpu.VMEM((_NBUF, _CW, _EMB), jnp.float32),
        pltpu.VMEM((_TROWS, _EMB), jnp.float32),
        pltpu.VMEM((_EMB,), jnp.float32),
        pltpu.VMEM((_EMB,), jnp.float32),
        pltpu.VMEM_SHARED((_VOCAB, _EMB), jnp.float32),
        pltpu.SemaphoreType.DMA((_NBUF,)),
        pltpu.SemaphoreType.DMA((_NBUF,)),
    ],
)
def _sc_embed(table_hbm, gamma_hbm, beta_hbm, idx_hbm, out_hbm,
              idx_v, bufs, tbl_v, gamma_v, beta_v, ntable_sp, gsems, wsems):
    sid = lax.axis_index("s")
    wid = sid * _NC + lax.axis_index("c")
    base = wid * _BPW

    start = lax.min(sid * _ROWS_PER_SUB, _VOCAB - _ROWS_PER_SUB)
    pltpu.sync_copy(gamma_hbm, gamma_v)
    pltpu.sync_copy(beta_hbm, beta_v)
    pltpu.sync_copy(idx_hbm.at[pl.ds(0, _L), pl.ds(wid * _BPW, _BPW)], idx_v)

    inv_emb = 1.0 / _EMB

    def ln_row(i, carry):
        x = [tbl_v[i, pl.ds(j * _LANES, _LANES)] for j in range(_NV)]
        tot = x[0]
        for j in range(1, _NV):
            tot = tot + x[j]
        mean = jnp.full((_LANES,), jnp.sum(tot)) * inv_emb
        c = [xj - mean for xj in x]
        sq = c[0] * c[0]
        for j in range(1, _NV):
            sq = sq + c[j] * c[j]
        var = jnp.full((_LANES,), jnp.sum(sq)) * inv_emb
        v = var + _EPS
        iv = lax.bitcast_convert_type(v, jnp.int32)
        y = lax.bitcast_convert_type(
            jnp.full((_LANES,), jnp.int32(0x5F3759DF)) - (iv >> 1), jnp.float32
        )
        for _ in range(4):
            y = y * (1.5 - 0.5 * v * y * y)
        for j in range(_NV):
            g = gamma_v[pl.ds(j * _LANES, _LANES)]
            b = beta_v[pl.ds(j * _LANES, _LANES)]
            tbl_v[i, pl.ds(j * _LANES, _LANES)] = c[j] * y * g + b
        return carry

    for h in range(_ROWS_PER_SUB // _TROWS):
        hs = start + h * _TROWS
        pltpu.sync_copy(table_hbm.at[pl.ds(hs, _TROWS)], tbl_v)
        lax.fori_loop(0, _TROWS, ln_row, 0)
        pltpu.sync_copy(tbl_v, ntable_sp.at[pl.ds(hs, _TROWS)])
    plsc.subcore_barrier()

    def chunk_refs(m, k):
        l = m // _CPL
        col = base + (m % _CPL) * _CW
        idx_sl = idx_v.at[l, pl.ds((m % _CPL) * _CW, _CW)]
        return ntable_sp.at[idx_sl], bufs.at[k], out_hbm.at[l, pl.ds(col, _CW)]

    def fire_gather(m, k):
        src, buf, _ = chunk_refs(m, k)
        pltpu.async_copy(src, buf, gsems.at[k])

    def drain_gather(m, k):
        src, buf, _ = chunk_refs(m, k)
        pltpu.make_async_copy(src, buf, gsems.at[k]).wait()

    def fire_write(m, k):
        _, buf, dst = chunk_refs(m, k)
        pltpu.async_copy(buf, dst, wsems.at[k])

    def wait_write(m, k):
        _, buf, dst = chunk_refs(m, k)
        pltpu.make_async_copy(buf, dst, wsems.at[k]).wait()

    for m in range(_AHEAD):
        fire_gather(m, m % _NBUF)

    def body(m, carry):
        k = m % _NBUF
        drain_gather(m, k)
        fire_write(m, k)

        nxt = m + _AHEAD
        kn = nxt % _NBUF

        @pl.when(nxt < _NCHUNK)
        def _():
            @pl.when(nxt >= _NBUF)
            def _():
                wait_write(nxt - _NBUF, kn)

            fire_gather(nxt, kn)

        return carry

    lax.fori_loop(0, _NCHUNK, body, 0)

    for t in range(_NBUF):
        m = _NCHUNK - _NBUF + t
        wait_write(m, m % _NBUF)


def kernel(indices, table, gamma, beta):
    out_lbe = _sc_embed(table, gamma, beta,
                        jnp.transpose(indices.astype(jnp.int32), (1, 0)))
    return jnp.transpose(out_lbe, (1, 0, 2))
